# baseline (speedup 1.0000x reference)
import functools

import jax
import jax.numpy as jnp
from jax import lax
from jax.experimental import pallas as pl
from jax.experimental.pallas import tpu as pltpu
from jax.experimental.pallas import tpu_sc as plsc

D_EMBED = 64
BATCH = 16384
MAX_TOKEN_LEN = 20
B_TOTAL = BATCH * MAX_TOKEN_LEN

NUM_CORES = 2
NUM_SUBCORES = 16
NW = NUM_CORES * NUM_SUBCORES
B_PER_W = B_TOTAL // NW
CHUNK = 128
NCH = B_PER_W // CHUNK


def _make_kernel():
    mesh = plsc.VectorSubcoreMesh(core_axis_name="c", subcore_axis_name="s")

    @functools.partial(
        pl.kernel,
        mesh=mesh,
        out_type=jax.ShapeDtypeStruct((B_TOTAL, D_EMBED), jnp.float32),
        scratch_types=[
            pltpu.VMEM((NCH, CHUNK), jnp.int32),
            pltpu.VMEM((CHUNK, D_EMBED), jnp.float32),
            pltpu.SemaphoreType.DMA,
        ],
        compiler_params=pltpu.CompilerParams(use_tc_tiling_on_sc=False),
    )
    def gather_kernel(idx_hbm, table_hbm, out_hbm, idx_v, rows_v, sem):
        wid = lax.axis_index("s") * NUM_CORES + lax.axis_index("c")
        base = wid * B_PER_W
        pltpu.sync_copy(idx_hbm.at[wid], idx_v)

        def body(j, carry):
            pltpu.async_copy(table_hbm.at[idx_v.at[j]], rows_v, sem).wait()
            pltpu.sync_copy(rows_v, out_hbm.at[pl.ds(base + j * CHUNK, CHUNK)])
            return carry

        lax.fori_loop(0, NCH, body, 0)

    return gather_kernel


_gather = _make_kernel()


@jax.jit
def kernel(scenario_tag_ids, tag_embedding_weight):
    idx = scenario_tag_ids.reshape(NW, NCH, CHUNK).astype(jnp.int32)
    out = _gather(idx, tag_embedding_weight)
    return out.reshape(BATCH, MAX_TOKEN_LEN, D_EMBED)


# speedup vs baseline: 1.0646x; 1.0646x over previous
import functools

import jax
import jax.numpy as jnp
from jax import lax
from jax.experimental import pallas as pl
from jax.experimental.pallas import tpu as pltpu
from jax.experimental.pallas import tpu_sc as plsc

D_EMBED = 64
BATCH = 16384
MAX_TOKEN_LEN = 20
B_TOTAL = BATCH * MAX_TOKEN_LEN

NUM_CORES = 2
NUM_SUBCORES = 16
NW = NUM_CORES * NUM_SUBCORES
B_PER_W = B_TOTAL // NW
CHUNK = 128
NCH = B_PER_W // CHUNK
GROUP = 4
NGRP = NCH // GROUP
NCH_TOTAL = B_TOTAL // CHUNK


def _make_kernel():
    mesh = plsc.VectorSubcoreMesh(core_axis_name="c", subcore_axis_name="s")

    @functools.partial(
        pl.kernel,
        mesh=mesh,
        out_type=jax.ShapeDtypeStruct((NCH_TOTAL, CHUNK, D_EMBED), jnp.float32),
        scratch_types=[
            pltpu.VMEM((NCH, CHUNK), jnp.int32),
            pltpu.VMEM((2, GROUP, CHUNK, D_EMBED), jnp.float32),
            pltpu.SemaphoreType.DMA,
            pltpu.SemaphoreType.DMA,
            pltpu.SemaphoreType.DMA,
            pltpu.SemaphoreType.DMA,
        ],
        compiler_params=pltpu.CompilerParams(use_tc_tiling_on_sc=False),
    )
    def gather_kernel(idx_hbm, table_hbm, out_hbm, idx_v, rows_v,
                      sem_g0, sem_g1, sem_w0, sem_w1):
        wid = lax.axis_index("s") * NUM_CORES + lax.axis_index("c")
        grp_base = wid * NGRP
        sem_g = (sem_g0, sem_g1)
        sem_w = (sem_w0, sem_w1)

        pltpu.sync_copy(idx_hbm.at[wid], idx_v)

        def issue_group(g, parity):
            for b in range(GROUP):
                pltpu.async_copy(
                    table_hbm.at[idx_v.at[g * GROUP + b]],
                    rows_v.at[parity].at[b],
                    sem_g[parity],
                )

        def wait_gathers(parity):
            pltpu.make_async_copy(
                out_hbm.at[pl.ds(0, GROUP)], rows_v.at[parity], sem_g[parity]
            ).wait()

        def out_copy(g, parity):
            pltpu.async_copy(
                rows_v.at[parity],
                out_hbm.at[pl.ds((grp_base + g) * GROUP, GROUP)],
                sem_w[parity],
            )

        def wait_out(parity):
            pltpu.make_async_copy(
                rows_v.at[parity], out_hbm.at[pl.ds(0, GROUP)], sem_w[parity]
            ).wait()

        issue_group(0, 0)
        issue_group(1, 1)
        wait_gathers(0)
        out_copy(0, 0)

        def step(t, parity_cur):
            wait_out(1 - parity_cur)
            issue_group(t + 1, 1 - parity_cur)
            wait_gathers(parity_cur)
            out_copy(t, parity_cur)

        def body(u, carry):
            step(2 * u + 1, 1)
            step(2 * u + 2, 0)
            return carry

        lax.fori_loop(0, (NGRP - 2) // 2, body, 0)

        wait_gathers(1)
        out_copy(NGRP - 1, 1)
        wait_out(0)
        wait_out(1)

    return gather_kernel


_gather = _make_kernel()


@jax.jit
def kernel(scenario_tag_ids, tag_embedding_weight):
    idx = scenario_tag_ids.reshape(NW, NCH, CHUNK).astype(jnp.int32)
    out = _gather(idx, tag_embedding_weight)
    return out.reshape(BATCH, MAX_TOKEN_LEN, D_EMBED)
